# baseline (speedup 1.0000x reference)
import functools

import jax
import jax.numpy as jnp
import numpy as np
from jax import lax
from jax.experimental import pallas as pl
from jax.experimental.pallas import tpu as pltpu
from jax.experimental.pallas import tpu_sc as plsc

_N_BINS = 10
_N_ROWS = 65536
_N_COLS = 1000
_BLK = 1024
_GRID = _N_ROWS // _BLK
_SUBL = _BLK // 128

_BOUNDS = np.linspace(0.0, 1.0, _N_BINS + 1)
_LO = [float(_BOUNDS[b]) for b in range(_N_BINS)]
_HI = [float(_BOUNDS[b + 1]) for b in range(_N_BINS)]

_N_WORKERS = 32
_CHUNK = _N_ROWS // _N_WORKERS
_NVEC = _CHUNK // 16
_ROWW = _N_BINS * 3 * 16


def _to_lane_major(v):
    k_row = lax.broadcasted_iota(jnp.int32, (_BLK, 128), 0)
    k_col = lax.broadcasted_iota(jnp.int32, (_BLK, 128), 1)
    m = jnp.where(k_row % 128 == k_col, 1.0, 0.0).astype(jnp.float32)
    a_row = lax.broadcasted_iota(jnp.int32, (_SUBL, _BLK), 0)
    a_col = lax.broadcasted_iota(jnp.int32, (_SUBL, _BLK), 1)
    a = jnp.where(a_col // 128 == a_row, 1.0, 0.0).astype(jnp.float32)
    return jax.lax.dot(a, v * m, precision=jax.lax.Precision.HIGHEST)


def _rowstats_body(logits_ref, out_ref):
    x = logits_ref[...]
    m = jnp.max(x, axis=1, keepdims=True)
    s = jnp.sum(jnp.exp(x), axis=1, keepdims=True)
    conf_lm = _to_lane_major(jnp.exp(m) / s)
    col = lax.broadcasted_iota(jnp.int32, x.shape, 1)
    pred = jnp.min(jnp.where(x == m, col, _N_COLS), axis=1, keepdims=True)
    pred_lm = _to_lane_major(pred.astype(jnp.float32))
    out_ref[...] = jnp.concatenate([conf_lm, pred_lm], axis=0)


def _rowstats(logits):
    return pl.pallas_call(
        _rowstats_body,
        grid=(_GRID,),
        in_specs=[
            pl.BlockSpec((_BLK, _N_COLS), lambda i: (i, 0)),
        ],
        out_specs=pl.BlockSpec((2 * _SUBL, 128), lambda i: (i, 0)),
        out_shape=jax.ShapeDtypeStruct((_GRID * 2 * _SUBL, 128), jnp.float32),
    )(logits)


def _binsum_body(stats_hbm, labels_hbm, out_hbm, conf_v, pred_v, lab_v, out_v):
    wid = lax.axis_index("s") * 2 + lax.axis_index("c")
    blk = (wid * _CHUNK) // _BLK
    off = (wid * _CHUNK) % _BLK
    base_c = blk * (2 * _BLK) + off
    base_p = base_c + _BLK
    pltpu.sync_copy(stats_hbm.at[pl.ds(base_c, _CHUNK)], conf_v)
    pltpu.sync_copy(stats_hbm.at[pl.ds(base_p, _CHUNK)], pred_v)
    pltpu.sync_copy(labels_hbm.at[pl.ds(wid * _CHUNK, _CHUNK)], lab_v)

    zeros = jnp.zeros((16,), jnp.float32)
    init = (zeros,) * (3 * _N_BINS)

    def body(i, accs):
        c = conf_v[pl.ds(i * 16, 16)]
        p = pred_v[pl.ds(i * 16, 16)]
        l = lab_v[pl.ds(i * 16, 16)].astype(jnp.float32)
        r = jnp.where(p == l, 1.0, 0.0).astype(jnp.float32)
        nxt = []
        for b in range(_N_BINS):
            mk = jnp.logical_and(c > _LO[b], c <= _HI[b])
            mf = jnp.where(mk, 1.0, 0.0).astype(jnp.float32)
            nxt.append(accs[3 * b] + mf)
            nxt.append(accs[3 * b + 1] + mf * r)
            nxt.append(accs[3 * b + 2] + mf * c)
        return tuple(nxt)

    accs = lax.fori_loop(0, _NVEC, body, init)
    for b in range(_N_BINS):
        out_v[pl.ds(b * 48, 16)] = accs[3 * b]
        out_v[pl.ds(b * 48 + 16, 16)] = accs[3 * b + 1]
        out_v[pl.ds(b * 48 + 32, 16)] = accs[3 * b + 2]
    pltpu.sync_copy(out_v, out_hbm.at[wid])


def _binsum(stats_flat, labels):
    mesh = plsc.VectorSubcoreMesh(core_axis_name="c", subcore_axis_name="s")
    fn = functools.partial(
        pl.kernel,
        mesh=mesh,
        out_type=jax.ShapeDtypeStruct((_N_WORKERS, _ROWW), jnp.float32),
        scratch_types=[
            pltpu.VMEM((_CHUNK,), jnp.float32),
            pltpu.VMEM((_CHUNK,), jnp.float32),
            pltpu.VMEM((_CHUNK,), jnp.int32),
            pltpu.VMEM((_ROWW,), jnp.float32),
        ],
    )(_binsum_body)
    return fn(stats_flat, labels)


def _ece_body(part_ref, train_ref, out_ref):
    p = part_ref[...]
    tr = train_ref[0, 0]
    ece = jnp.float32(0.0)
    for b in range(_N_BINS):
        cnt = jnp.sum(p[:, b * 48:b * 48 + 16])
        cor = jnp.sum(p[:, b * 48 + 16:b * 48 + 32]) * tr
        cnf = jnp.sum(p[:, b * 48 + 32:b * 48 + 48])
        denom = jnp.maximum(cnt, 1.0)
        contrib = jnp.abs(cnf / denom - cor / denom) * (cnt / _N_ROWS)
        ece = ece + jnp.where(cnt > 0.0, contrib, 0.0)
    out_ref[...] = jnp.broadcast_to(ece, (1, 1))


def _ece(parts, train2d):
    return pl.pallas_call(
        _ece_body,
        in_specs=[
            pl.BlockSpec((_N_WORKERS, _ROWW), lambda: (0, 0)),
            pl.BlockSpec(memory_space=pltpu.SMEM),
        ],
        out_shape=jax.ShapeDtypeStruct((1, 1), jnp.float32),
    )(parts, train2d)


def kernel(logits, labels, train):
    labels_i = labels.astype(jnp.int32)
    train2d = jnp.asarray(train, jnp.float32).reshape(1, 1)
    stats = _rowstats(logits)
    parts = _binsum(stats.reshape(_GRID * 2 * _BLK), labels_i)
    return _ece(parts, train2d).reshape(1)


# speedup vs baseline: 1.0226x; 1.0226x over previous
import functools

import jax
import jax.numpy as jnp
import numpy as np
from jax import lax
from jax.experimental import pallas as pl
from jax.experimental.pallas import tpu as pltpu
from jax.experimental.pallas import tpu_sc as plsc

_N_BINS = 10
_N_ROWS = 65536
_N_COLS = 1000
_BLK = 2048
_GRID = _N_ROWS // _BLK
_SUBL = _BLK // 128

_BOUNDS = np.linspace(0.0, 1.0, _N_BINS + 1)
_LO = [float(_BOUNDS[b]) for b in range(_N_BINS)]
_HI = [float(_BOUNDS[b + 1]) for b in range(_N_BINS)]

_N_WORKERS = 32
_CHUNK = _N_ROWS // _N_WORKERS
_NVEC = _CHUNK // 16
_ROWW = _N_BINS * 3 * 16


def _to_lane_major(v):
    k_row = lax.broadcasted_iota(jnp.int32, (_BLK, 128), 0)
    k_col = lax.broadcasted_iota(jnp.int32, (_BLK, 128), 1)
    m = jnp.where(k_row % 128 == k_col, 1.0, 0.0).astype(jnp.float32)
    a_row = lax.broadcasted_iota(jnp.int32, (_SUBL, _BLK), 0)
    a_col = lax.broadcasted_iota(jnp.int32, (_SUBL, _BLK), 1)
    a = jnp.where(a_col // 128 == a_row, 1.0, 0.0).astype(jnp.float32)
    return jax.lax.dot(a, v * m, precision=jax.lax.Precision.HIGHEST)


def _rowstats_body(logits_ref, out_ref):
    x = logits_ref[...]
    m = jnp.max(x, axis=1, keepdims=True)
    s = jnp.sum(jnp.exp(x), axis=1, keepdims=True)
    conf_lm = _to_lane_major(jnp.exp(m) / s)
    col = lax.broadcasted_iota(jnp.int32, x.shape, 1)
    pred = jnp.min(jnp.where(x == m, col, _N_COLS), axis=1, keepdims=True)
    pred_lm = _to_lane_major(pred.astype(jnp.float32))
    out_ref[...] = jnp.concatenate([conf_lm, pred_lm], axis=0)


def _rowstats(logits):
    return pl.pallas_call(
        _rowstats_body,
        grid=(_GRID,),
        in_specs=[
            pl.BlockSpec((_BLK, _N_COLS), lambda i: (i, 0)),
        ],
        out_specs=pl.BlockSpec((2 * _SUBL, 128), lambda i: (i, 0)),
        out_shape=jax.ShapeDtypeStruct((_GRID * 2 * _SUBL, 128), jnp.float32),
    )(logits)


def _binsum_body(stats_hbm, labels_hbm, out_hbm, conf_v, pred_v, lab_v, out_v):
    wid = lax.axis_index("s") * 2 + lax.axis_index("c")
    blk = (wid * _CHUNK) // _BLK
    off = (wid * _CHUNK) % _BLK
    base_c = blk * (2 * _BLK) + off
    base_p = base_c + _BLK
    pltpu.sync_copy(stats_hbm.at[pl.ds(base_c, _CHUNK)], conf_v)
    pltpu.sync_copy(stats_hbm.at[pl.ds(base_p, _CHUNK)], pred_v)
    pltpu.sync_copy(labels_hbm.at[pl.ds(wid * _CHUNK, _CHUNK)], lab_v)

    zeros = jnp.zeros((16,), jnp.float32)
    init = (zeros,) * (3 * _N_BINS)

    def body(i, accs):
        c = conf_v[pl.ds(i * 16, 16)]
        p = pred_v[pl.ds(i * 16, 16)]
        l = lab_v[pl.ds(i * 16, 16)].astype(jnp.float32)
        r = jnp.where(p == l, 1.0, 0.0).astype(jnp.float32)
        nxt = []
        for b in range(_N_BINS):
            mk = jnp.logical_and(c > _LO[b], c <= _HI[b])
            mf = jnp.where(mk, 1.0, 0.0).astype(jnp.float32)
            nxt.append(accs[3 * b] + mf)
            nxt.append(accs[3 * b + 1] + mf * r)
            nxt.append(accs[3 * b + 2] + mf * c)
        return tuple(nxt)

    accs = lax.fori_loop(0, _NVEC, body, init)
    for b in range(_N_BINS):
        out_v[pl.ds(b * 48, 16)] = accs[3 * b]
        out_v[pl.ds(b * 48 + 16, 16)] = accs[3 * b + 1]
        out_v[pl.ds(b * 48 + 32, 16)] = accs[3 * b + 2]
    pltpu.sync_copy(out_v, out_hbm.at[wid])


def _binsum(stats_flat, labels):
    mesh = plsc.VectorSubcoreMesh(core_axis_name="c", subcore_axis_name="s")
    fn = functools.partial(
        pl.kernel,
        mesh=mesh,
        out_type=jax.ShapeDtypeStruct((_N_WORKERS, _ROWW), jnp.float32),
        scratch_types=[
            pltpu.VMEM((_CHUNK,), jnp.float32),
            pltpu.VMEM((_CHUNK,), jnp.float32),
            pltpu.VMEM((_CHUNK,), jnp.int32),
            pltpu.VMEM((_ROWW,), jnp.float32),
        ],
    )(_binsum_body)
    return fn(stats_flat, labels)


def _ece_body(part_ref, train_ref, out_ref):
    p = part_ref[...]
    tr = train_ref[0, 0]
    ece = jnp.float32(0.0)
    for b in range(_N_BINS):
        cnt = jnp.sum(p[:, b * 48:b * 48 + 16])
        cor = jnp.sum(p[:, b * 48 + 16:b * 48 + 32]) * tr
        cnf = jnp.sum(p[:, b * 48 + 32:b * 48 + 48])
        denom = jnp.maximum(cnt, 1.0)
        contrib = jnp.abs(cnf / denom - cor / denom) * (cnt / _N_ROWS)
        ece = ece + jnp.where(cnt > 0.0, contrib, 0.0)
    out_ref[...] = jnp.broadcast_to(ece, (1, 1))


def _ece(parts, train2d):
    return pl.pallas_call(
        _ece_body,
        in_specs=[
            pl.BlockSpec((_N_WORKERS, _ROWW), lambda: (0, 0)),
            pl.BlockSpec(memory_space=pltpu.SMEM),
        ],
        out_shape=jax.ShapeDtypeStruct((1, 1), jnp.float32),
    )(parts, train2d)


def kernel(logits, labels, train):
    labels_i = labels.astype(jnp.int32)
    train2d = jnp.asarray(train, jnp.float32).reshape(1, 1)
    stats = _rowstats(logits)
    parts = _binsum(stats.reshape(_GRID * 2 * _BLK), labels_i)
    return _ece(parts, train2d).reshape(1)


# speedup vs baseline: 1.0253x; 1.0026x over previous
import functools

import jax
import jax.numpy as jnp
import numpy as np
from jax import lax
from jax.experimental import pallas as pl
from jax.experimental.pallas import tpu as pltpu
from jax.experimental.pallas import tpu_sc as plsc

_N_BINS = 10
_N_ROWS = 65536
_N_COLS = 1000
_BLK = 2048
_GRID = _N_ROWS // _BLK
_SUBL = _BLK // 128

_BOUNDS = np.linspace(0.0, 1.0, _N_BINS + 1)
_LO = [float(_BOUNDS[b]) for b in range(_N_BINS)]
_HI = [float(_BOUNDS[b + 1]) for b in range(_N_BINS)]

_N_WORKERS = 32
_CHUNK = _N_ROWS // _N_WORKERS
_NVEC = _CHUNK // 16
_ROWW = _N_BINS * 3 * 16


def _to_lane_major(v):
    k_row = lax.broadcasted_iota(jnp.int32, (_BLK, 128), 0)
    k_col = lax.broadcasted_iota(jnp.int32, (_BLK, 128), 1)
    m = jnp.where(k_row % 128 == k_col, 1.0, 0.0).astype(jnp.float32)
    a_row = lax.broadcasted_iota(jnp.int32, (_SUBL, _BLK), 0)
    a_col = lax.broadcasted_iota(jnp.int32, (_SUBL, _BLK), 1)
    a = jnp.where(a_col // 128 == a_row, 1.0, 0.0).astype(jnp.float32)
    return jax.lax.dot(a, v * m, precision=jax.lax.Precision.HIGHEST)


def _rowstats_body(logits_ref, out_ref):
    x = logits_ref[...]
    m = jnp.max(x, axis=1, keepdims=True)
    s = jnp.sum(jnp.exp(x), axis=1, keepdims=True)
    conf_lm = _to_lane_major(jnp.exp(m) / s)
    col = lax.broadcasted_iota(jnp.int32, x.shape, 1)
    pred = jnp.min(jnp.where(x == m, col, _N_COLS), axis=1, keepdims=True)
    pred_lm = _to_lane_major(pred.astype(jnp.float32))
    out_ref[0:_SUBL, :] = conf_lm
    out_ref[_SUBL:2 * _SUBL, :] = pred_lm


def _rowstats(logits):
    return pl.pallas_call(
        _rowstats_body,
        grid=(_GRID,),
        in_specs=[
            pl.BlockSpec((_BLK, _N_COLS), lambda i: (i, 0)),
        ],
        out_specs=pl.BlockSpec((2 * _SUBL, 128), lambda i: (i, 0)),
        out_shape=jax.ShapeDtypeStruct((_GRID * 2 * _SUBL, 128), jnp.float32),
    )(logits)


def _binsum_body(stats_hbm, labels_hbm, out_hbm, conf_v, pred_v, lab_v, out_v):
    wid = lax.axis_index("s") * 2 + lax.axis_index("c")
    blk = (wid * _CHUNK) // _BLK
    off = (wid * _CHUNK) % _BLK
    base_c = blk * (2 * _BLK) + off
    base_p = base_c + _BLK
    pltpu.sync_copy(stats_hbm.at[pl.ds(base_c, _CHUNK)], conf_v)
    pltpu.sync_copy(stats_hbm.at[pl.ds(base_p, _CHUNK)], pred_v)
    pltpu.sync_copy(labels_hbm.at[pl.ds(wid * _CHUNK, _CHUNK)], lab_v)

    zeros = jnp.zeros((16,), jnp.float32)
    init = (zeros,) * (3 * _N_BINS)

    def body(i, accs):
        c = conf_v[pl.ds(i * 16, 16)]
        p = pred_v[pl.ds(i * 16, 16)]
        l = lab_v[pl.ds(i * 16, 16)].astype(jnp.float32)
        r = jnp.where(p == l, 1.0, 0.0).astype(jnp.float32)
        nxt = []
        for b in range(_N_BINS):
            mk = jnp.logical_and(c > _LO[b], c <= _HI[b])
            mf = jnp.where(mk, 1.0, 0.0).astype(jnp.float32)
            nxt.append(accs[3 * b] + mf)
            nxt.append(accs[3 * b + 1] + mf * r)
            nxt.append(accs[3 * b + 2] + mf * c)
        return tuple(nxt)

    accs = lax.fori_loop(0, _NVEC, body, init)
    for b in range(_N_BINS):
        out_v[pl.ds(b * 48, 16)] = accs[3 * b]
        out_v[pl.ds(b * 48 + 16, 16)] = accs[3 * b + 1]
        out_v[pl.ds(b * 48 + 32, 16)] = accs[3 * b + 2]
    pltpu.sync_copy(out_v, out_hbm.at[wid])


def _binsum(stats_flat, labels):
    mesh = plsc.VectorSubcoreMesh(core_axis_name="c", subcore_axis_name="s")
    fn = functools.partial(
        pl.kernel,
        mesh=mesh,
        out_type=jax.ShapeDtypeStruct((_N_WORKERS, _ROWW), jnp.float32),
        scratch_types=[
            pltpu.VMEM((_CHUNK,), jnp.float32),
            pltpu.VMEM((_CHUNK,), jnp.float32),
            pltpu.VMEM((_CHUNK,), jnp.int32),
            pltpu.VMEM((_ROWW,), jnp.float32),
        ],
    )(_binsum_body)
    return fn(stats_flat, labels)


def _ece_body(part_ref, train_ref, out_ref):
    p = part_ref[...]
    tr = train_ref[0, 0]
    ece = jnp.float32(0.0)
    for b in range(_N_BINS):
        cnt = jnp.sum(p[:, b * 48:b * 48 + 16])
        cor = jnp.sum(p[:, b * 48 + 16:b * 48 + 32]) * tr
        cnf = jnp.sum(p[:, b * 48 + 32:b * 48 + 48])
        denom = jnp.maximum(cnt, 1.0)
        contrib = jnp.abs(cnf / denom - cor / denom) * (cnt / _N_ROWS)
        ece = ece + jnp.where(cnt > 0.0, contrib, 0.0)
    out_ref[...] = jnp.broadcast_to(ece, (1, 1))


def _ece(parts, train2d):
    return pl.pallas_call(
        _ece_body,
        in_specs=[
            pl.BlockSpec((_N_WORKERS, _ROWW), lambda: (0, 0)),
            pl.BlockSpec(memory_space=pltpu.SMEM),
        ],
        out_shape=jax.ShapeDtypeStruct((1, 1), jnp.float32),
    )(parts, train2d)


def kernel(logits, labels, train):
    labels_i = labels.astype(jnp.int32)
    train2d = jnp.asarray(train, jnp.float32).reshape(1, 1)
    stats = _rowstats(logits)
    parts = _binsum(stats.reshape(_GRID * 2 * _BLK), labels_i)
    return _ece(parts, train2d).reshape(1)
